# baseline (speedup 1.0000x reference)
import functools

import jax
import jax.numpy as jnp
from jax import lax
from jax.experimental import pallas as pl
from jax.experimental.pallas import tpu as pltpu
from jax.experimental.pallas import tpu_sc as plsc

VOCAB = 1000000
DIM = 64
MAXLEN = 200
BATCH = 4096
B = BATCH * MAXLEN

NC, NS, L = 2, 16, 16
NW = NC * NS
B_PER_W = B // NW
CHUNK = 800
N_CHUNKS = B_PER_W // CHUNK
PERIODS = CHUNK // MAXLEN


def _body(x_hbm, tok_hbm, pos_hbm, out_hbm, idx_v, rows_v, pos_v, sem):
    wid = lax.axis_index("s") * NC + lax.axis_index("c")
    pltpu.sync_copy(pos_hbm, pos_v)

    @pl.loop(0, N_CHUNKS)
    def _chunk(j):
        base = wid * B_PER_W + j * CHUNK
        pltpu.sync_copy(x_hbm.at[pl.ds(base, CHUNK)], idx_v)
        pltpu.async_copy(tok_hbm.at[idx_v], rows_v, sem).wait()

        @pl.loop(0, CHUNK)
        def _row(i):
            r = lax.rem(i, MAXLEN)
            for c in range(DIM // L):
                s = pl.ds(c * L, L)
                rows_v[i, s] = rows_v[i, s] + pos_v[r, s]

        pltpu.sync_copy(rows_v, out_hbm.at[pl.ds(base, CHUNK)])


@jax.jit
def _run(x_flat, token_table, pos_table):
    mesh = plsc.VectorSubcoreMesh(core_axis_name="c", subcore_axis_name="s")
    return pl.kernel(
        _body,
        out_type=jax.ShapeDtypeStruct((B, DIM), jnp.float32),
        mesh=mesh,
        scratch_types=[
            pltpu.VMEM((CHUNK,), jnp.int32),
            pltpu.VMEM((CHUNK, DIM), jnp.float32),
            pltpu.VMEM((MAXLEN, DIM), jnp.float32),
            pltpu.SemaphoreType.DMA,
        ],
        compiler_params=pltpu.CompilerParams(use_tc_tiling_on_sc=False),
    )(x_flat, token_table, pos_table)


def kernel(x, token_table, pos_table):
    x_flat = x.reshape(-1).astype(jnp.int32)
    out = _run(x_flat, token_table, pos_table)
    return out.reshape(BATCH, MAXLEN, DIM)


# speedup vs baseline: 1.3859x; 1.3859x over previous
import jax
import jax.numpy as jnp
from jax import lax
from jax.experimental import pallas as pl
from jax.experimental.pallas import tpu as pltpu
from jax.experimental.pallas import tpu_sc as plsc

VOCAB = 1000000
DIM = 64
MAXLEN = 200
BATCH = 4096

NC, NS, L = 2, 16, 16
NW = NC * NS
CB = 512
NBC = BATCH // CB
NT = MAXLEN * NBC
TPW = NT // NW
RPW = TPW
WPB = NW // NBC


def _body(xT_hbm, tok_hbm, pos_hbm, out_hbm,
          idx0, idx1, buf0, buf1, pos_v, sg0, sg1, ss0, ss1):
    wid = lax.axis_index("s") * NC + lax.axis_index("c")
    bc = wid // WPB
    r0 = (wid % WPB) * RPW
    bbase = bc * CB
    idxs, bufs = (idx0, idx1), (buf0, buf1)
    sgs, sss = (sg0, sg1), (ss0, ss1)

    pltpu.sync_copy(pos_hbm, pos_v)

    def gather_start(t, slot):
        r = r0 + t
        pltpu.sync_copy(xT_hbm.at[r, pl.ds(bbase, CB)], idxs[slot])
        pltpu.async_copy(tok_hbm.at[idxs[slot]], bufs[slot], sgs[slot])

    def gather_wait(slot):
        pltpu.make_async_copy(tok_hbm.at[idxs[slot]], bufs[slot],
                              sgs[slot]).wait()

    def store_start(t, slot):
        r = r0 + t
        pltpu.async_copy(bufs[slot], out_hbm.at[pl.ds(bbase, CB), r],
                         sss[slot])

    def store_wait(slot):
        pltpu.make_async_copy(bufs[slot], out_hbm.at[pl.ds(bbase, CB), 0],
                              sss[slot]).wait()

    gather_start(0, 0)

    @pl.loop(0, TPW, step=2)
    def _pair(k):
        for b in range(2):
            cur = k + b
            nb = 1 - b

            @pl.when(cur + 1 < TPW)
            def _():
                @pl.when(cur >= 1)
                def _():
                    store_wait(nb)
                gather_start(cur + 1, nb)

            gather_wait(b)

            r = r0 + cur
            pv = [pos_v[r, pl.ds(c * L, L)] for c in range(DIM // L)]
            buf = bufs[b]

            @pl.loop(0, CB, unroll=8)
            def _row(i):
                for c in range(DIM // L):
                    s = pl.ds(c * L, L)
                    buf[i, s] = buf[i, s] + pv[c]

            store_start(cur, b)

    store_wait(0)
    store_wait(1)


@jax.jit
def _run(xT, token_table, pos_table):
    mesh = plsc.VectorSubcoreMesh(core_axis_name="c", subcore_axis_name="s")
    return pl.kernel(
        _body,
        out_type=jax.ShapeDtypeStruct((BATCH, MAXLEN, DIM), jnp.float32),
        mesh=mesh,
        scratch_types=[
            pltpu.VMEM((CB,), jnp.int32),
            pltpu.VMEM((CB,), jnp.int32),
            pltpu.VMEM((CB, DIM), jnp.float32),
            pltpu.VMEM((CB, DIM), jnp.float32),
            pltpu.VMEM((MAXLEN, DIM), jnp.float32),
            pltpu.SemaphoreType.DMA,
            pltpu.SemaphoreType.DMA,
            pltpu.SemaphoreType.DMA,
            pltpu.SemaphoreType.DMA,
        ],
        compiler_params=pltpu.CompilerParams(use_tc_tiling_on_sc=False),
    )(xT, token_table, pos_table)


def kernel(x, token_table, pos_table):
    xT = jnp.swapaxes(x.astype(jnp.int32), 0, 1)
    return _run(xT, token_table, pos_table)
